# baseline (speedup 1.0000x reference)
import functools

import jax
import jax.numpy as jnp
from jax import lax
from jax.experimental import pallas as pl
from jax.experimental.pallas import tpu as pltpu
from jax.experimental.pallas import tpu_sc as plsc

N = 10000
E = 320000
H = 128
NC = 2
NS = 16
NW = NC * NS
EPW = E // NW
K = 50
WINS = EPW // K
NB = 3
WMAIN = WINS - 2
NP = 10112
RPS = NP // NS


def _sc_segsum(g, src, dst, zeros):
    mesh = plsc.VectorSubcoreMesh(core_axis_name="c", subcore_axis_name="s")

    @functools.partial(
        pl.kernel,
        out_type=jax.ShapeDtypeStruct((NC, NP, H), jnp.float32),
        mesh=mesh,
        scratch_types=[
            pltpu.VMEM((NB, 1, K), jnp.int32),
            pltpu.VMEM((WINS, K), jnp.int32),
            pltpu.VMEM((NB, K, H), jnp.float32),
            pltpu.VMEM_SHARED((NP, H), jnp.float32),
            pltpu.SemaphoreType.DMA((NB,)),
            pltpu.SemaphoreType.DMA,
            pltpu.SemaphoreType.DMA,
            pltpu.SemaphoreType.DMA((NB,)),
            pltpu.SemaphoreType.DMA((NB,)),
        ],
    )
    def k(g_hbm, src_hbm, dst_hbm, z_hbm, out_hbm, isring, dsts, bufs, acc,
          isems, dsem, zsem, gsems, ssems):
        cid = lax.axis_index("c")
        sid = lax.axis_index("s")
        wid = sid * NC + cid
        row0 = sid * RPS
        wbase = wid * WINS

        for i in range(NB):
            pltpu.async_copy(src_hbm.at[wbase + i], isring.at[i],
                             isems.at[i])
        dc = pltpu.async_copy(dst_hbm.at[wid], dsts, dsem)
        zc = pltpu.async_copy(z_hbm, acc.at[pl.ds(row0, RPS)], zsem)

        for b in range(NB - 1):
            pltpu.make_async_copy(src_hbm.at[wbase + b], isring.at[b],
                                  isems.at[b]).wait()
            pltpu.async_copy(g_hbm.at[isring.at[b].at[0]], bufs.at[b],
                             gsems.at[b])
        dc.wait()
        zc.wait()
        plsc.subcore_barrier()

        def visit(w, b):
            bn = (b + 2) % NB

            @pl.when(w + 2 < WINS)
            def _():
                @pl.when(w >= 1)
                def _():
                    pltpu.make_async_copy(bufs.at[bn], acc.at[dsts.at[w]],
                                          ssems.at[bn]).wait()
                pltpu.make_async_copy(src_hbm.at[wbase + w + 2],
                                      isring.at[bn], isems.at[bn]).wait()
                pltpu.async_copy(g_hbm.at[isring.at[bn].at[0]],
                                 bufs.at[bn], gsems.at[bn])

            pltpu.make_async_copy(g_hbm.at[isring.at[b].at[0]],
                                  bufs.at[b], gsems.at[b]).wait()
            pltpu.async_copy(bufs.at[b], acc.at[dsts.at[w]],
                             ssems.at[b], add=True)

            @pl.when(w + NB < WINS)
            def _():
                pltpu.async_copy(src_hbm.at[wbase + w + NB],
                                 isring.at[b], isems.at[b])

        @pl.loop(0, WMAIN, step=NB)
        def _(w0):
            for u in range(NB):
                visit(w0 + u, u)

        visit(WMAIN, WMAIN % NB)
        visit(WMAIN + 1, (WMAIN + 1) % NB)
        for b in range(NB):
            pltpu.make_async_copy(bufs.at[b], acc.at[dsts.at[0]],
                                  ssems.at[b]).wait()
        plsc.subcore_barrier()
        pltpu.sync_copy(acc.at[pl.ds(row0, RPS)],
                        out_hbm.at[cid].at[pl.ds(row0, RPS)])

    return k(g, src, dst, zeros)


def _dot_t(a, b):
    return lax.dot_general(a, b, (((1,), (1,)), ((), ())),
                           preferred_element_type=jnp.float32)


def _tc_pre(x, wr, wo, br):
    def body(x_ref, wr_ref, wo_ref, br_ref, g_ref, r_ref):
        xv = x_ref[...]
        g_ref[...] = _dot_t(xv, wr_ref[...])
        r_ref[...] = _dot_t(xv, wo_ref[...]) + br_ref[...]

    return pl.pallas_call(
        body,
        out_shape=(jax.ShapeDtypeStruct((N, H), jnp.float32),
                   jax.ShapeDtypeStruct((N, H), jnp.float32)),
    )(x, wr, wo, br.reshape(1, H))


def _tc_mid(p, r_prev, wr, wo, br):
    def body(p_ref, rp_ref, wr_ref, wo_ref, br_ref, g_ref, r_ref):
        h = jnp.maximum(p_ref[0, :N, :] + p_ref[1, :N, :] + rp_ref[...], 0.0)
        g_ref[...] = _dot_t(h, wr_ref[...])
        r_ref[...] = _dot_t(h, wo_ref[...]) + br_ref[...]

    return pl.pallas_call(
        body,
        out_shape=(jax.ShapeDtypeStruct((N, H), jnp.float32),
                   jax.ShapeDtypeStruct((N, H), jnp.float32)),
    )(p, r_prev, wr, wo, br.reshape(1, H))


def _tc_fin(p, r_prev, lin_w, lin_b):
    def body(p_ref, rp_ref, lw_ref, lb_ref, o_ref):
        h = jnp.maximum(p_ref[0, :N, :] + p_ref[1, :N, :] + rp_ref[...], 0.0)
        emb = jnp.sum(h, axis=0, keepdims=True) * (1.0 / N)
        o_ref[...] = _dot_t(emb, lw_ref[...]) + lb_ref[...]

    c = lin_w.shape[0]
    return pl.pallas_call(
        body,
        out_shape=jax.ShapeDtypeStruct((1, c), jnp.float32),
    )(p, r_prev, lin_w, lin_b.reshape(1, c))


def kernel(x, edge_index, W_rel1, b_rel1, W_root1, W_rel2, b_rel2, W_root2,
           W_rel3, b_rel3, W_root3, lin_W, lin_b):
    src = edge_index[0].reshape(NW * WINS, 1, K)
    dst = edge_index[1].reshape(NW, WINS, K)
    zeros = jnp.zeros((RPS, H), jnp.float32)

    g1, r1 = _tc_pre(x, W_rel1, W_root1, b_rel1)
    p1 = _sc_segsum(g1, src, dst, zeros)
    g2, r2 = _tc_mid(p1, r1, W_rel2, W_root2, b_rel2)
    p2 = _sc_segsum(g2, src, dst, zeros)
    g3, r3 = _tc_mid(p2, r2, W_rel3, W_root3, b_rel3)
    p3 = _sc_segsum(g3, src, dst, zeros)
    return _tc_fin(p3, r3, lin_W, lin_b)


# speedup vs baseline: 1.2295x; 1.2295x over previous
import functools

import jax
import jax.numpy as jnp
from jax import lax
from jax.experimental import pallas as pl
from jax.experimental.pallas import tpu as pltpu
from jax.experimental.pallas import tpu_sc as plsc

N = 10000
E = 320000
H = 128
NC = 2
NS = 16
NW = NC * NS
EPW = E // NW
K = 80
WINS = EPW // K
NB = 3
WMAIN = WINS - 2
NP = 10112
RPS = NP // NS


def _sc_segsum(g, src, dst, zeros):
    mesh = plsc.VectorSubcoreMesh(core_axis_name="c", subcore_axis_name="s")

    @functools.partial(
        pl.kernel,
        out_type=jax.ShapeDtypeStruct((NC, NP, H), jnp.float32),
        mesh=mesh,
        scratch_types=[
            pltpu.VMEM((NB, 1, K), jnp.int32),
            pltpu.VMEM((WINS, K), jnp.int32),
            pltpu.VMEM((NB, K, H), jnp.float32),
            pltpu.VMEM_SHARED((NP, H), jnp.float32),
            pltpu.SemaphoreType.DMA((NB,)),
            pltpu.SemaphoreType.DMA,
            pltpu.SemaphoreType.DMA,
            pltpu.SemaphoreType.DMA((NB,)),
            pltpu.SemaphoreType.DMA((NB,)),
        ],
    )
    def k(g_hbm, src_hbm, dst_hbm, z_hbm, out_hbm, isring, dsts, bufs, acc,
          isems, dsem, zsem, gsems, ssems):
        cid = lax.axis_index("c")
        sid = lax.axis_index("s")
        wid = sid * NC + cid
        row0 = sid * RPS
        wbase = wid * WINS

        for i in range(NB):
            pltpu.async_copy(src_hbm.at[wbase + i], isring.at[i],
                             isems.at[i])
        dc = pltpu.async_copy(dst_hbm.at[wid], dsts, dsem)
        zc = pltpu.async_copy(z_hbm, acc.at[pl.ds(row0, RPS)], zsem)

        for b in range(NB - 1):
            pltpu.make_async_copy(src_hbm.at[wbase + b], isring.at[b],
                                  isems.at[b]).wait()
            pltpu.async_copy(g_hbm.at[isring.at[b].at[0]], bufs.at[b],
                             gsems.at[b])
        dc.wait()
        zc.wait()
        plsc.subcore_barrier()

        def visit(w, b):
            bn = (b + 2) % NB

            @pl.when(w + 2 < WINS)
            def _():
                @pl.when(w >= 1)
                def _():
                    pltpu.make_async_copy(bufs.at[bn], acc.at[dsts.at[w]],
                                          ssems.at[bn]).wait()
                pltpu.make_async_copy(src_hbm.at[wbase + w + 2],
                                      isring.at[bn], isems.at[bn]).wait()
                pltpu.async_copy(g_hbm.at[isring.at[bn].at[0]],
                                 bufs.at[bn], gsems.at[bn])

            pltpu.make_async_copy(g_hbm.at[isring.at[b].at[0]],
                                  bufs.at[b], gsems.at[b]).wait()
            pltpu.async_copy(bufs.at[b], acc.at[dsts.at[w]],
                             ssems.at[b], add=True)

            @pl.when(w + NB < WINS)
            def _():
                pltpu.async_copy(src_hbm.at[wbase + w + NB],
                                 isring.at[b], isems.at[b])

        @pl.loop(0, WMAIN, step=NB)
        def _(w0):
            for u in range(NB):
                visit(w0 + u, u)

        visit(WMAIN, WMAIN % NB)
        visit(WMAIN + 1, (WMAIN + 1) % NB)
        for b in range(NB):
            pltpu.make_async_copy(bufs.at[b], acc.at[dsts.at[0]],
                                  ssems.at[b]).wait()
        plsc.subcore_barrier()
        pltpu.sync_copy(acc.at[pl.ds(row0, RPS)],
                        out_hbm.at[cid].at[pl.ds(row0, RPS)])

    return k(g, src, dst, zeros)


def _dot_t(a, b):
    return lax.dot_general(a, b, (((1,), (1,)), ((), ())),
                           preferred_element_type=jnp.float32)


def _tc_pre(x, wr, wo, br):
    def body(x_ref, wr_ref, wo_ref, br_ref, g_ref, r_ref):
        xv = x_ref[...]
        g_ref[...] = _dot_t(xv, wr_ref[...])
        r_ref[...] = _dot_t(xv, wo_ref[...]) + br_ref[...]

    return pl.pallas_call(
        body,
        out_shape=(jax.ShapeDtypeStruct((N, H), jnp.float32),
                   jax.ShapeDtypeStruct((N, H), jnp.float32)),
    )(x, wr, wo, br.reshape(1, H))


def _tc_mid(p, r_prev, wr, wo, br):
    def body(p_ref, rp_ref, wr_ref, wo_ref, br_ref, g_ref, r_ref):
        h = jnp.maximum(p_ref[0, :N, :] + p_ref[1, :N, :] + rp_ref[...], 0.0)
        g_ref[...] = _dot_t(h, wr_ref[...])
        r_ref[...] = _dot_t(h, wo_ref[...]) + br_ref[...]

    return pl.pallas_call(
        body,
        out_shape=(jax.ShapeDtypeStruct((N, H), jnp.float32),
                   jax.ShapeDtypeStruct((N, H), jnp.float32)),
    )(p, r_prev, wr, wo, br.reshape(1, H))


def _tc_fin(p, r_prev, lin_w, lin_b):
    def body(p_ref, rp_ref, lw_ref, lb_ref, o_ref):
        h = jnp.maximum(p_ref[0, :N, :] + p_ref[1, :N, :] + rp_ref[...], 0.0)
        emb = jnp.sum(h, axis=0, keepdims=True) * (1.0 / N)
        o_ref[...] = _dot_t(emb, lw_ref[...]) + lb_ref[...]

    c = lin_w.shape[0]
    return pl.pallas_call(
        body,
        out_shape=jax.ShapeDtypeStruct((1, c), jnp.float32),
    )(p, r_prev, lin_w, lin_b.reshape(1, c))


def kernel(x, edge_index, W_rel1, b_rel1, W_root1, W_rel2, b_rel2, W_root2,
           W_rel3, b_rel3, W_root3, lin_W, lin_b):
    src = edge_index[0].reshape(NW * WINS, 1, K)
    dst = edge_index[1].reshape(NW, WINS, K)
    zeros = jnp.zeros((RPS, H), jnp.float32)

    g1, r1 = _tc_pre(x, W_rel1, W_root1, b_rel1)
    p1 = _sc_segsum(g1, src, dst, zeros)
    g2, r2 = _tc_mid(p1, r1, W_rel2, W_root2, b_rel2)
    p2 = _sc_segsum(g2, src, dst, zeros)
    g3, r3 = _tc_mid(p2, r2, W_rel3, W_root3, b_rel3)
    p3 = _sc_segsum(g3, src, dst, zeros)
    return _tc_fin(p3, r3, lin_W, lin_b)


# speedup vs baseline: 1.2380x; 1.0069x over previous
import functools

import jax
import jax.numpy as jnp
from jax import lax
from jax.experimental import pallas as pl
from jax.experimental.pallas import tpu as pltpu
from jax.experimental.pallas import tpu_sc as plsc

N = 10000
E = 320000
H = 128
NC = 2
NS = 16
NW = NC * NS
EPW = E // NW
K = 80
WINS = EPW // K
NB = 3
WMAIN = WINS - 2
NP = 10112
RPS = NP // NS


def _sc_segsum(g, src, dst, zeros):
    mesh = plsc.VectorSubcoreMesh(core_axis_name="c", subcore_axis_name="s")

    @functools.partial(
        pl.kernel,
        out_type=jax.ShapeDtypeStruct((NC, NP, H), jnp.float32),
        mesh=mesh,
        scratch_types=[
            pltpu.VMEM((NB, K), jnp.int32),
            pltpu.VMEM((NB, K), jnp.int32),
            pltpu.VMEM((NB, K, H), jnp.float32),
            pltpu.VMEM_SHARED((NP, H), jnp.float32),
            pltpu.SemaphoreType.DMA((NB,)),
            pltpu.SemaphoreType.DMA((NB,)),
            pltpu.SemaphoreType.DMA,
            pltpu.SemaphoreType.DMA((NB,)),
            pltpu.SemaphoreType.DMA((NB,)),
        ],
    )
    def k(g_hbm, sr, dr, z_hbm, out_hbm, isring, dring, bufs, acc,
          isems, dsems, zsem, gsems, ssems):
        cid = lax.axis_index("c")
        sid = lax.axis_index("s")
        wid = sid * NC + cid
        row0 = sid * RPS
        ebase = wid * EPW

        for i in range(NB):
            pltpu.async_copy(sr.at[pl.ds(ebase + i * K, K)], isring.at[i],
                             isems.at[i])
        for i in range(NB - 1):
            pltpu.async_copy(dr.at[pl.ds(ebase + i * K, K)], dring.at[i],
                             dsems.at[i])
        zc = pltpu.async_copy(z_hbm, acc.at[pl.ds(row0, RPS)], zsem)

        for b in range(NB - 1):
            pltpu.make_async_copy(sr.at[pl.ds(ebase, K)], isring.at[b],
                                  isems.at[b]).wait()
            pltpu.async_copy(g_hbm.at[isring.at[b]], bufs.at[b],
                             gsems.at[b])
        zc.wait()
        plsc.subcore_barrier()

        def visit(w, b):
            bn = (b + 2) % NB

            @pl.when(w + 2 < WINS)
            def _():
                @pl.when(w >= 1)
                def _():
                    pltpu.make_async_copy(bufs.at[bn], acc.at[dring.at[b]],
                                          ssems.at[bn]).wait()
                pltpu.async_copy(dr.at[pl.ds(ebase + (w + 2) * K, K)],
                                 dring.at[bn], dsems.at[bn])
                pltpu.make_async_copy(sr.at[pl.ds(ebase, K)],
                                      isring.at[bn], isems.at[bn]).wait()
                pltpu.async_copy(g_hbm.at[isring.at[bn]], bufs.at[bn],
                                 gsems.at[bn])

            pltpu.make_async_copy(g_hbm.at[isring.at[b]], bufs.at[b],
                                  gsems.at[b]).wait()
            pltpu.make_async_copy(dr.at[pl.ds(ebase, K)], dring.at[b],
                                  dsems.at[b]).wait()
            pltpu.async_copy(bufs.at[b], acc.at[dring.at[b]],
                             ssems.at[b], add=True)

            @pl.when(w + NB < WINS)
            def _():
                pltpu.async_copy(sr.at[pl.ds(ebase + (w + NB) * K, K)],
                                 isring.at[b], isems.at[b])

        @pl.loop(0, WMAIN, step=NB)
        def _(w0):
            for u in range(NB):
                visit(w0 + u, u)

        visit(WMAIN, WMAIN % NB)
        visit(WMAIN + 1, (WMAIN + 1) % NB)
        for b in range(NB):
            pltpu.make_async_copy(bufs.at[b], acc.at[dring.at[0]],
                                  ssems.at[b]).wait()
        plsc.subcore_barrier()
        pltpu.sync_copy(acc.at[pl.ds(row0, RPS)],
                        out_hbm.at[cid].at[pl.ds(row0, RPS)])

    return k(g, src, dst, zeros)


def _dot_t(a, b):
    return lax.dot_general(a, b, (((1,), (1,)), ((), ())),
                           preferred_element_type=jnp.float32)


def _tc_pre(x, wr, wo, br):
    def body(x_ref, wr_ref, wo_ref, br_ref, g_ref, r_ref):
        xv = x_ref[...]
        g_ref[...] = _dot_t(xv, wr_ref[...])
        r_ref[...] = _dot_t(xv, wo_ref[...]) + br_ref[...]

    return pl.pallas_call(
        body,
        out_shape=(jax.ShapeDtypeStruct((N, H), jnp.float32),
                   jax.ShapeDtypeStruct((N, H), jnp.float32)),
    )(x, wr, wo, br.reshape(1, H))


def _tc_mid(p, r_prev, wr, wo, br):
    def body(p_ref, rp_ref, wr_ref, wo_ref, br_ref, g_ref, r_ref):
        h = jnp.maximum(p_ref[0, :N, :] + p_ref[1, :N, :] + rp_ref[...], 0.0)
        g_ref[...] = _dot_t(h, wr_ref[...])
        r_ref[...] = _dot_t(h, wo_ref[...]) + br_ref[...]

    return pl.pallas_call(
        body,
        out_shape=(jax.ShapeDtypeStruct((N, H), jnp.float32),
                   jax.ShapeDtypeStruct((N, H), jnp.float32)),
    )(p, r_prev, wr, wo, br.reshape(1, H))


def _tc_fin(p, r_prev, lin_w, lin_b):
    def body(p_ref, rp_ref, lw_ref, lb_ref, o_ref):
        h = jnp.maximum(p_ref[0, :N, :] + p_ref[1, :N, :] + rp_ref[...], 0.0)
        emb = jnp.sum(h, axis=0, keepdims=True) * (1.0 / N)
        o_ref[...] = _dot_t(emb, lw_ref[...]) + lb_ref[...]

    c = lin_w.shape[0]
    return pl.pallas_call(
        body,
        out_shape=jax.ShapeDtypeStruct((1, c), jnp.float32),
    )(p, r_prev, lin_w, lin_b.reshape(1, c))


def kernel(x, edge_index, W_rel1, b_rel1, W_root1, W_rel2, b_rel2, W_root2,
           W_rel3, b_rel3, W_root3, lin_W, lin_b):
    zeros = jnp.zeros((RPS, H), jnp.float32)
    src = edge_index[0]
    dst = edge_index[1]

    g1, r1 = _tc_pre(x, W_rel1, W_root1, b_rel1)
    p1 = _sc_segsum(g1, src, dst, zeros)
    g2, r2 = _tc_mid(p1, r1, W_rel2, W_root2, b_rel2)
    p2 = _sc_segsum(g2, src, dst, zeros)
    g3, r3 = _tc_mid(p2, r2, W_rel3, W_root3, b_rel3)
    p3 = _sc_segsum(g3, src, dst, zeros)
    return _tc_fin(p3, r3, lin_W, lin_b)


# speedup vs baseline: 1.3017x; 1.0514x over previous
import functools

import jax
import jax.numpy as jnp
from jax import lax
from jax.experimental import pallas as pl
from jax.experimental.pallas import tpu as pltpu
from jax.experimental.pallas import tpu_sc as plsc

N = 10000
E = 320000
H = 128
NC = 2
NS = 16
NW = NC * NS
EPW = E // NW
K = 80
WINS = EPW // K
NB = 3
WMAIN = WINS - 2
NP = 10112
RPS = NP // NS


def _sc_segsum(g, src, dst, zeros):
    mesh = plsc.VectorSubcoreMesh(core_axis_name="c", subcore_axis_name="s")

    @functools.partial(
        pl.kernel,
        out_type=jax.ShapeDtypeStruct((NC, NP, H), jnp.float32),
        mesh=mesh,
        scratch_types=[
            pltpu.VMEM((NB, K), jnp.int32),
            pltpu.VMEM((NB, K), jnp.int32),
            pltpu.VMEM((NB, K, H), jnp.float32),
            pltpu.VMEM_SHARED((NP, H), jnp.float32),
            pltpu.SemaphoreType.DMA((NB,)),
            pltpu.SemaphoreType.DMA((NB,)),
            pltpu.SemaphoreType.DMA,
            pltpu.SemaphoreType.DMA((NB,)),
            pltpu.SemaphoreType.DMA((NB,)),
        ],
    )
    def k(g_hbm, sr, dr, z_hbm, out_hbm, isring, dring, bufs, acc,
          isems, dsems, zsem, gsems, ssems):
        cid = lax.axis_index("c")
        sid = lax.axis_index("s")
        wid = sid * NC + cid
        row0 = sid * RPS
        ebase = wid * EPW

        for i in range(NB):
            pltpu.async_copy(sr.at[pl.ds(ebase + i * K, K)], isring.at[i],
                             isems.at[i])
        for i in range(NB - 1):
            pltpu.async_copy(dr.at[pl.ds(ebase + i * K, K)], dring.at[i],
                             dsems.at[i])
        zc = pltpu.async_copy(z_hbm, acc.at[pl.ds(row0, RPS)], zsem)

        for b in range(NB - 1):
            pltpu.make_async_copy(sr.at[pl.ds(ebase, K)], isring.at[b],
                                  isems.at[b]).wait()
            pltpu.async_copy(g_hbm.at[isring.at[b]], bufs.at[b],
                             gsems.at[b])
        zc.wait()
        plsc.subcore_barrier()

        def visit(w, b):
            bn = (b + 2) % NB

            @pl.when(w + 2 < WINS)
            def _():
                @pl.when(w >= 1)
                def _():
                    pltpu.make_async_copy(bufs.at[bn], acc.at[dring.at[b]],
                                          ssems.at[bn]).wait()
                pltpu.async_copy(dr.at[pl.ds(ebase + (w + 2) * K, K)],
                                 dring.at[bn], dsems.at[bn])
                pltpu.make_async_copy(sr.at[pl.ds(ebase, K)],
                                      isring.at[bn], isems.at[bn]).wait()
                pltpu.async_copy(g_hbm.at[isring.at[bn]], bufs.at[bn],
                                 gsems.at[bn])

            pltpu.make_async_copy(g_hbm.at[isring.at[b]], bufs.at[b],
                                  gsems.at[b]).wait()
            pltpu.make_async_copy(dr.at[pl.ds(ebase, K)], dring.at[b],
                                  dsems.at[b]).wait()
            pltpu.async_copy(bufs.at[b], acc.at[dring.at[b]],
                             ssems.at[b], add=True)

            @pl.when(w + NB < WINS)
            def _():
                pltpu.async_copy(sr.at[pl.ds(ebase + (w + NB) * K, K)],
                                 isring.at[b], isems.at[b])

        @pl.loop(0, WMAIN, step=NB)
        def _(w0):
            for u in range(NB):
                visit(w0 + u, u)

        visit(WMAIN, WMAIN % NB)
        visit(WMAIN + 1, (WMAIN + 1) % NB)
        for b in range(NB):
            pltpu.make_async_copy(bufs.at[b], acc.at[dring.at[0]],
                                  ssems.at[b]).wait()
        plsc.subcore_barrier()
        pltpu.sync_copy(acc.at[pl.ds(row0, RPS)],
                        out_hbm.at[cid].at[pl.ds(row0, RPS)])

    return k(g, src, dst, zeros)


def _dot_t(a, b):
    return lax.dot_general(a, b, (((1,), (1,)), ((), ())),
                           preferred_element_type=jnp.float32)


def _tc_pre(x, wr, wo, br, edge_index):
    def body(x_ref, wr_ref, wo_ref, br_ref, e_ref, g_ref, r_ref, src_ref,
             dst_ref):
        xv = x_ref[...]
        g_ref[...] = _dot_t(xv, wr_ref[...])
        r_ref[...] = _dot_t(xv, wo_ref[...]) + br_ref[...]
        src_ref[...] = e_ref[0, :]
        dst_ref[...] = e_ref[1, :]

    return pl.pallas_call(
        body,
        out_shape=(jax.ShapeDtypeStruct((N, H), jnp.float32),
                   jax.ShapeDtypeStruct((N, H), jnp.float32),
                   jax.ShapeDtypeStruct((E,), jnp.int32),
                   jax.ShapeDtypeStruct((E,), jnp.int32)),
    )(x, wr, wo, br.reshape(1, H), edge_index)


def _tc_mid(p, r_prev, wr, wo, br):
    def body(p_ref, rp_ref, wr_ref, wo_ref, br_ref, g_ref, r_ref):
        h = jnp.maximum(p_ref[0, :N, :] + p_ref[1, :N, :] + rp_ref[...], 0.0)
        g_ref[...] = _dot_t(h, wr_ref[...])
        r_ref[...] = _dot_t(h, wo_ref[...]) + br_ref[...]

    return pl.pallas_call(
        body,
        out_shape=(jax.ShapeDtypeStruct((N, H), jnp.float32),
                   jax.ShapeDtypeStruct((N, H), jnp.float32)),
    )(p, r_prev, wr, wo, br.reshape(1, H))


def _tc_fin(p, r_prev, lin_w, lin_b):
    def body(p_ref, rp_ref, lw_ref, lb_ref, o_ref):
        h = jnp.maximum(p_ref[0, :N, :] + p_ref[1, :N, :] + rp_ref[...], 0.0)
        emb = jnp.sum(h, axis=0, keepdims=True) * (1.0 / N)
        o_ref[...] = _dot_t(emb, lw_ref[...]) + lb_ref[...]

    c = lin_w.shape[0]
    return pl.pallas_call(
        body,
        out_shape=jax.ShapeDtypeStruct((1, c), jnp.float32),
    )(p, r_prev, lin_w, lin_b.reshape(1, c))


def kernel(x, edge_index, W_rel1, b_rel1, W_root1, W_rel2, b_rel2, W_root2,
           W_rel3, b_rel3, W_root3, lin_W, lin_b):
    zeros = jnp.zeros((RPS, H), jnp.float32)

    g1, r1, src, dst = _tc_pre(x, W_rel1, W_root1, b_rel1, edge_index)
    p1 = _sc_segsum(g1, src, dst, zeros)
    g2, r2 = _tc_mid(p1, r1, W_rel2, W_root2, b_rel2)
    p2 = _sc_segsum(g2, src, dst, zeros)
    g3, r3 = _tc_mid(p2, r2, W_rel3, W_root3, b_rel3)
    p3 = _sc_segsum(g3, src, dst, zeros)
    return _tc_fin(p3, r3, lin_W, lin_b)
